# baseline (speedup 1.0000x reference)
import functools

import jax
import jax.numpy as jnp
from jax import lax
from jax.experimental import pallas as pl
from jax.experimental.pallas import tpu as pltpu
from jax.experimental.pallas import tpu_sc as plsc

N = 10000
E = 160000
D = 128
DE = 16

NC = 2
NS = 16
NW = NC * NS

NCH0 = 40
NCH1 = 40
CHUNK = 128
E0 = NS * NCH0 * CHUNK
NCHMAX = max(NCH0, NCH1)
NROWS_X = E0 // CHUNK + 15 * NCH1 + NCHMAX
NROWS_X = max((NROWS_X + 7) // 8 * 8, 1296)
EPADX = NROWS_X * CHUNK

CHUNKE = 125
NCHE = 40
EPTE = NCHE * CHUNKE

NPAD = 10112
ROWS_PT = NPAD // NS


def _tc1_body(xa_ref, xs_ref,
              wse1, wde1, wse2, wde2, wse3, wde3,
              s1, d1, s2, d2, s3, d3):
    xa = xa_ref[...]
    xs = xs_ref[...]
    dot = functools.partial(jnp.dot, preferred_element_type=jnp.float32)
    s1[...] = dot(xa, wse1[...])
    d1[...] = dot(xa, wde1[...])
    s2[...] = dot(xa, wse2[...])
    d2[...] = dot(xs, wde2[...])
    s3[...] = dot(xs, wse3[...])
    d3[...] = dot(xa, wde3[...])


def _tc2_body(a1, b1, a2, b2, a3, b3, xa_ref, xs_ref,
              wm1, we1, ws1, wm2, we2, ws2, wm3, we3, ws3,
              nap, nsta):
    xa = xa_ref[...]
    xs = xs_ref[...]
    dot = functools.partial(jnp.dot, preferred_element_type=jnp.float32)

    def dstd(a, b, wm, we, ws, x):
        acc = dot(a[0] + a[1], wm[...]) + dot(b[0] + b[1], we[...])
        return jnp.maximum(acc + dot(x, ws[...]), 0.0)

    d_apap = dstd(a1[...], b1[...], wm1, we1, ws1, xa)
    d_apsta = dstd(a2[...], b2[...], wm2, we2, ws2, xs)
    d_staap = dstd(a3[...], b3[...], wm3, we3, ws3, xa)
    nap[...] = 0.5 * (d_apap + d_staap)
    nsta[...] = d_apsta


def _tc3_body(sg1, dg1, e1, sg2, dg2, e2, sg3, dg3, e3,
              wee1, wee2, wee3, o1, o2, o3):
    dot = functools.partial(jnp.dot, preferred_element_type=jnp.float32)
    o1[...] = jnp.maximum(sg1[...] + dg1[...] + dot(e1[...], wee1[...]), 0.0)
    o2[...] = jnp.maximum(sg2[...] + dg2[...] + dot(e2[...], wee2[...]), 0.0)
    o3[...] = jnp.maximum(sg3[...] + dg3[...] + dot(e3[...], wee3[...]), 0.0)


def kernel(x_ap, x_sta, edge_index_apap, edge_index_apsta, edge_index_staap,
           e_apap, e_apsta, e_staap,
           W_msg_apap, W_e_apap, W_self_apap, W_se_apap, W_de_apap, W_ee_apap,
           W_msg_apsta, W_e_apsta, W_self_apsta, W_se_apsta, W_de_apsta, W_ee_apsta,
           W_msg_staap, W_e_staap, W_self_staap, W_se_staap, W_de_staap, W_ee_staap):
    f32 = jnp.float32

    nb = 10
    blk = N // nb
    row_spec = pl.BlockSpec((blk, D), lambda i: (i, 0))
    w_spec = pl.BlockSpec((D, DE), lambda i: (0, 0))
    out16 = pl.BlockSpec((blk, DE), lambda i: (i, 0))
    s1, d1, s2, d2, s3, d3 = pl.pallas_call(
        _tc1_body,
        grid=(nb,),
        in_specs=[row_spec, row_spec] + [w_spec] * 6,
        out_specs=[out16] * 6,
        out_shape=[jax.ShapeDtypeStruct((N, DE), f32)] * 6,
    )(x_ap, x_sta,
      W_se_apap, W_de_apap, W_se_apsta, W_de_apsta, W_se_staap, W_de_staap)

    pad_n = EPADX - E

    pad_src = (jnp.arange(pad_n, dtype=jnp.int32) * 53) % N
    pad_dst = N + (jnp.arange(pad_n, dtype=jnp.int32) % (NPAD - N))

    def prep_idx_x(ei):
        src = jnp.concatenate([ei[0], pad_src])
        dst = jnp.concatenate([ei[1], pad_dst])
        return src.reshape(NROWS_X, CHUNK), dst.reshape(NROWS_X, CHUNK)

    def prep_idx_e(ei):
        return (ei[0].reshape(NW, NCHE, CHUNKE),
                ei[1].reshape(NW, NCHE, CHUNKE))

    sx1, dx1 = prep_idx_x(edge_index_apap)
    sx2, dx2 = prep_idx_x(edge_index_apsta)
    sx3, dx3 = prep_idx_x(edge_index_staap)
    se1, de1 = prep_idx_e(edge_index_apap)
    se2, de2 = prep_idx_e(edge_index_apsta)
    se3, de3 = prep_idx_e(edge_index_staap)
    zeros_a = jnp.zeros((ROWS_PT, D), f32)
    zeros_b = jnp.zeros((ROWS_PT, DE), f32)

    mesh = plsc.VectorSubcoreMesh(core_axis_name="c", subcore_axis_name="s")

    def make_p1():
        def body(xa, xs, si1, di1, si2, di2, si3, di3, za,
                 a1_out, a2_out, a3_out,
                 a_acc, src_v, dst_v, xb, gx0, gx1, tx0, tx1):
            gxs = (gx0, gx1)
            txs = (tx0, tx1)
            c = lax.axis_index("c")
            s = lax.axis_index("s")
            row0 = s * ROWS_PT
            rowbase = jnp.where(c == 0, s * NCH0, E0 // CHUNK + s * NCH1)
            nch = jnp.where(c == 0, NCH0, NCH1)

            def do_rel(x_src, src_i, dst_i, a_out, tag):
                with jax.named_scope("p1_zero" + tag):
                    pltpu.sync_copy(src_i.at[pl.ds(rowbase, NCHMAX)], src_v)
                    pltpu.sync_copy(dst_i.at[pl.ds(rowbase, NCHMAX)], dst_v)
                    pltpu.sync_copy(za, a_acc.at[pl.ds(row0, ROWS_PT)])
                    plsc.subcore_barrier()

                def gather(j, b):
                    return pltpu.make_async_copy(
                        x_src.at[src_v.at[j]], xb.at[b], gxs[b])

                def scat(j, b):
                    return pltpu.make_async_copy(
                        xb.at[b], a_acc.at[dst_v.at[j]], txs[b])

                gather(0, 0).start()

                @pl.loop(0, nch, step=2)
                def _(j):
                    for b in (0, 1):
                        jj = j + b
                        gather(jj, b).wait()

                        @pl.when(jj > 0)
                        def _():
                            scat(jj - 1, 1 - b).wait()

                        scat(jj, b).start(add=True)

                        @pl.when(jj + 1 < nch)
                        def _():
                            gather(jj + 1, 1 - b).start()

                with jax.named_scope("p1_tail" + tag):
                    scat(nch - 1, 1).wait()
                    plsc.subcore_barrier()
                with jax.named_scope("p1_out" + tag):
                    pltpu.sync_copy(a_acc.at[pl.ds(row0, ROWS_PT)],
                                    a_out.at[c, pl.ds(row0, ROWS_PT)])

            do_rel(xa, si1, di1, a1_out, "_r0")
            do_rel(xa, si2, di2, a2_out, "_r1")
            do_rel(xs, si3, di3, a3_out, "_r2")

        return pl.kernel(
            body,
            out_type=[jax.ShapeDtypeStruct((NC, NPAD, D), f32)] * 3,
            mesh=mesh,
            scratch_types=[
                pltpu.VMEM_SHARED((NPAD, D), f32),
                pltpu.VMEM((NCHMAX, CHUNK), jnp.int32),
                pltpu.VMEM((NCHMAX, CHUNK), jnp.int32),
                pltpu.VMEM((2, CHUNK, D), f32),
            ] + [pltpu.SemaphoreType.DMA] * 4,
        )

    def make_p2():
        def body(st1, dt1, ep1, st2, dt2, ep2, st3, dt3, ep3,
                 si1, di1, si2, di2, si3, di3, zb,
                 b1_out, b2_out, b3_out, sg1_o, dg1_o, sg2_o, dg2_o, sg3_o, dg3_o,
                 b_acc, src_v, dst_v, eb, sb, db, ge, gs, gd, te, ts, td):
            c = lax.axis_index("c")
            s = lax.axis_index("s")
            wid = c * NS + s
            row0 = s * ROWS_PT

            def do_rel(s_tab, d_tab, e_raw, src_i, dst_i, b_out, sg_out, dg_out):
                pltpu.sync_copy(src_i.at[wid], src_v)
                pltpu.sync_copy(dst_i.at[wid], dst_v)
                pltpu.sync_copy(zb, b_acc.at[pl.ds(row0, ROWS_PT)])
                plsc.subcore_barrier()

                def gathers(j, b):
                    erow = wid * EPTE + j * CHUNKE
                    return (
                        pltpu.make_async_copy(
                            e_raw.at[pl.ds(erow, CHUNKE)], eb.at[b], ge),
                        pltpu.make_async_copy(
                            s_tab.at[src_v.at[j]], sb.at[b], gs),
                        pltpu.make_async_copy(
                            d_tab.at[dst_v.at[j]], db.at[b], gd),
                    )

                def stores(j, b):
                    erow = wid * EPTE + j * CHUNKE
                    return (
                        (pltpu.make_async_copy(
                            eb.at[b], b_acc.at[dst_v.at[j]], te), True),
                        (pltpu.make_async_copy(
                            sb.at[b], sg_out.at[pl.ds(erow, CHUNKE)], ts), False),
                        (pltpu.make_async_copy(
                            db.at[b], dg_out.at[pl.ds(erow, CHUNKE)], td), False),
                    )

                for dsc in gathers(0, 0):
                    dsc.start()

                @pl.loop(0, NCHE, step=2)
                def _(j):
                    for b in (0, 1):
                        jj = j + b
                        for dsc in gathers(jj, b):
                            dsc.wait()

                        @pl.when(jj > 0)
                        def _():
                            for dsc, _add in stores(jj - 1, 1 - b):
                                dsc.wait()

                        for dsc, _add in stores(jj, b):
                            dsc.start(add=_add)

                        @pl.when(jj + 1 < NCHE)
                        def _():
                            for dsc in gathers(jj + 1, 1 - b):
                                dsc.start()

                for dsc, _add in stores(NCHE - 1, (NCHE - 1) & 1):
                    dsc.wait()
                plsc.subcore_barrier()
                pltpu.sync_copy(b_acc.at[pl.ds(row0, ROWS_PT)],
                                b_out.at[c, pl.ds(row0, ROWS_PT)])

            do_rel(st1, dt1, ep1, si1, di1, b1_out, sg1_o, dg1_o)
            do_rel(st2, dt2, ep2, si2, di2, b2_out, sg2_o, dg2_o)
            do_rel(st3, dt3, ep3, si3, di3, b3_out, sg3_o, dg3_o)

        return pl.kernel(
            body,
            out_type=[jax.ShapeDtypeStruct((NC, NPAD, DE), f32)] * 3
                     + [jax.ShapeDtypeStruct((E, DE), f32)] * 6,
            mesh=mesh,
            compiler_params=pltpu.CompilerParams(use_tc_tiling_on_sc=False),
            scratch_types=[
                pltpu.VMEM_SHARED((NPAD, DE), f32),
                pltpu.VMEM((NCHE, CHUNKE), jnp.int32),
                pltpu.VMEM((NCHE, CHUNKE), jnp.int32),
                pltpu.VMEM((2, CHUNKE, DE), f32),
                pltpu.VMEM((2, CHUNKE, DE), f32),
                pltpu.VMEM((2, CHUNKE, DE), f32),
            ] + [pltpu.SemaphoreType.DMA] * 6,
        )

    a1o, a2o, a3o = make_p1()(x_ap, x_sta, sx1, dx1, sx2, dx2, sx3, dx3,
                              zeros_a)
    (s1, d1, e1g, s2, d2, e2g, s3, d3, e3g, a1o) = lax.optimization_barrier(
        (s1, d1, e_apap, s2, d2, e_apsta, s3, d3, e_staap, a1o))
    (b1o, b2o, b3o, sg1, dg1, sg2, dg2, sg3, dg3) = make_p2()(
        s1, d1, e1g, s2, d2, e2g, s3, d3, e3g,
        se1, de1, se2, de2, se3, de3, zeros_b)

    a_spec = pl.BlockSpec((NC, blk, D), lambda i: (0, i, 0))
    b_spec = pl.BlockSpec((NC, blk, DE), lambda i: (0, i, 0))
    wdd = pl.BlockSpec((D, D), lambda i: (0, 0))
    wed = pl.BlockSpec((DE, D), lambda i: (0, 0))
    outD = pl.BlockSpec((blk, D), lambda i: (i, 0))
    nap, nsta = pl.pallas_call(
        _tc2_body,
        grid=(nb,),
        in_specs=[a_spec, b_spec, a_spec, b_spec, a_spec, b_spec,
                  row_spec, row_spec,
                  wdd, wed, wdd, wdd, wed, wdd, wdd, wed, wdd],
        out_specs=[outD, outD],
        out_shape=[jax.ShapeDtypeStruct((N, D), f32)] * 2,
    )(a1o, b1o, a2o, b2o, a3o, b3o, x_ap, x_sta,
      W_msg_apap, W_e_apap, W_self_apap,
      W_msg_apsta, W_e_apsta, W_self_apsta,
      W_msg_staap, W_e_staap, W_self_staap)

    E8 = E // 8
    eb = 2000
    neb = E8 // eb
    eg_spec = pl.BlockSpec((eb, D), lambda i: (i, 0))
    wee_spec = pl.BlockSpec((D, D), lambda i: (0, 0))
    eye8 = jnp.eye(8, dtype=f32)
    pk = lambda a: a.reshape(E8, D)
    eo1, eo2, eo3 = pl.pallas_call(
        _tc3_body,
        grid=(neb,),
        in_specs=[eg_spec] * 9 + [wee_spec] * 3,
        out_specs=[eg_spec] * 3,
        out_shape=[jax.ShapeDtypeStruct((E8, D), f32)] * 3,
    )(pk(sg1), pk(dg1), pk(e_apap), pk(sg2), pk(dg2), pk(e_apsta),
      pk(sg3), pk(dg3), pk(e_staap),
      jnp.kron(eye8, W_ee_apap), jnp.kron(eye8, W_ee_apsta),
      jnp.kron(eye8, W_ee_staap))

    return (nap, nsta, eo1.reshape(E, DE), eo2.reshape(E, DE),
            eo3.reshape(E, DE))


# speedup vs baseline: 1.2652x; 1.2652x over previous
import functools

import jax
import jax.numpy as jnp
from jax import lax
from jax.experimental import pallas as pl
from jax.experimental.pallas import tpu as pltpu
from jax.experimental.pallas import tpu_sc as plsc

N = 10000
E = 160000
D = 128
DE = 16

NC = 2
NS = 16
NW = NC * NS

NCH0 = 40
NCH1 = 40
CHUNK = 128
E0 = NS * NCH0 * CHUNK
NCHMAX = max(NCH0, NCH1)
NROWS_X = E0 // CHUNK + 15 * NCH1 + NCHMAX
NROWS_X = max((NROWS_X + 7) // 8 * 8, 1296)
EPADX = NROWS_X * CHUNK

CHUNKE = 125
NCHE = 40
EPTE = NCHE * CHUNKE

NPAD = 10112
ROWS_PT = NPAD // NS


def _tc1_body(xa_ref, xs_ref,
              wse1, wde1, wse2, wde2, wse3, wde3,
              s1, d1, s2, d2, s3, d3):
    xa = xa_ref[...]
    xs = xs_ref[...]
    dot = functools.partial(jnp.dot, preferred_element_type=jnp.float32)
    s1[...] = dot(xa, wse1[...])
    d1[...] = dot(xa, wde1[...])
    s2[...] = dot(xa, wse2[...])
    d2[...] = dot(xs, wde2[...])
    s3[...] = dot(xs, wse3[...])
    d3[...] = dot(xa, wde3[...])


def _tc2_body(a1, b1, a2, b2, a3, b3, xa_ref, xs_ref,
              wm1, we1, ws1, wm2, we2, ws2, wm3, we3, ws3,
              nap, nsta):
    xa = xa_ref[...]
    xs = xs_ref[...]
    dot = functools.partial(jnp.dot, preferred_element_type=jnp.float32)

    def dstd(a, b, wm, we, ws, x):
        a32 = a[0].astype(jnp.float32) + a[1].astype(jnp.float32)
        acc = dot(a32, wm[...]) + dot(b[0] + b[1], we[...])
        return jnp.maximum(acc + dot(x, ws[...]), 0.0)

    d_apap = dstd(a1[...], b1[...], wm1, we1, ws1, xa)
    d_apsta = dstd(a2[...], b2[...], wm2, we2, ws2, xs)
    d_staap = dstd(a3[...], b3[...], wm3, we3, ws3, xa)
    nap[...] = 0.5 * (d_apap + d_staap)
    nsta[...] = d_apsta


def _tc3_body(sg1, dg1, e1, sg2, dg2, e2, sg3, dg3, e3,
              wee1, wee2, wee3, o1, o2, o3):
    dot = functools.partial(jnp.dot, preferred_element_type=jnp.float32)
    o1[...] = jnp.maximum(sg1[...] + dg1[...] + dot(e1[...], wee1[...]), 0.0)
    o2[...] = jnp.maximum(sg2[...] + dg2[...] + dot(e2[...], wee2[...]), 0.0)
    o3[...] = jnp.maximum(sg3[...] + dg3[...] + dot(e3[...], wee3[...]), 0.0)


def kernel(x_ap, x_sta, edge_index_apap, edge_index_apsta, edge_index_staap,
           e_apap, e_apsta, e_staap,
           W_msg_apap, W_e_apap, W_self_apap, W_se_apap, W_de_apap, W_ee_apap,
           W_msg_apsta, W_e_apsta, W_self_apsta, W_se_apsta, W_de_apsta, W_ee_apsta,
           W_msg_staap, W_e_staap, W_self_staap, W_se_staap, W_de_staap, W_ee_staap):
    f32 = jnp.float32

    nb = 10
    blk = N // nb
    row_spec = pl.BlockSpec((blk, D), lambda i: (i, 0))
    w_spec = pl.BlockSpec((D, DE), lambda i: (0, 0))
    out16 = pl.BlockSpec((blk, DE), lambda i: (i, 0))
    s1, d1, s2, d2, s3, d3 = pl.pallas_call(
        _tc1_body,
        grid=(nb,),
        in_specs=[row_spec, row_spec] + [w_spec] * 6,
        out_specs=[out16] * 6,
        out_shape=[jax.ShapeDtypeStruct((N, DE), f32)] * 6,
    )(x_ap, x_sta,
      W_se_apap, W_de_apap, W_se_apsta, W_de_apsta, W_se_staap, W_de_staap)

    pad_n = EPADX - E

    pad_src = (jnp.arange(pad_n, dtype=jnp.int32) * 53) % N
    pad_dst = N + (jnp.arange(pad_n, dtype=jnp.int32) % (NPAD - N))

    def prep_idx_x(ei):
        src = jnp.concatenate([ei[0], pad_src])
        dst = jnp.concatenate([ei[1], pad_dst])
        return src.reshape(NROWS_X, CHUNK), dst.reshape(NROWS_X, CHUNK)

    def prep_idx_e(ei):
        return (ei[0].reshape(NW, NCHE, CHUNKE),
                ei[1].reshape(NW, NCHE, CHUNKE))

    sx1, dx1 = prep_idx_x(edge_index_apap)
    sx2, dx2 = prep_idx_x(edge_index_apsta)
    sx3, dx3 = prep_idx_x(edge_index_staap)
    se1, de1 = prep_idx_e(edge_index_apap)
    se2, de2 = prep_idx_e(edge_index_apsta)
    se3, de3 = prep_idx_e(edge_index_staap)
    zeros_a = jnp.zeros((ROWS_PT, D), f32)
    zeros_b = jnp.zeros((ROWS_PT, DE), f32)

    mesh = plsc.VectorSubcoreMesh(core_axis_name="c", subcore_axis_name="s")

    def make_p1():
        def body(xa, xs, si1, di1, si2, di2, si3, di3, za,
                 a1_out, a2_out, a3_out,
                 a_acc, src_v, dst_v, xb, gx0, gx1, tx0, tx1):
            gxs = (gx0, gx1)
            txs = (tx0, tx1)
            c = lax.axis_index("c")
            s = lax.axis_index("s")
            row0 = s * ROWS_PT
            rowbase = jnp.where(c == 0, s * NCH0, E0 // CHUNK + s * NCH1)
            nch = jnp.where(c == 0, NCH0, NCH1)

            def do_rel(x_src, src_i, dst_i, a_out):
                pltpu.sync_copy(src_i.at[pl.ds(rowbase, NCHMAX)], src_v)
                pltpu.sync_copy(dst_i.at[pl.ds(rowbase, NCHMAX)], dst_v)
                pltpu.sync_copy(za, a_acc.at[pl.ds(row0, ROWS_PT)])
                plsc.subcore_barrier()

                def gather(j, b):
                    return pltpu.make_async_copy(
                        x_src.at[src_v.at[j]], xb.at[b], gxs[b])

                def scat(j, b):
                    return pltpu.make_async_copy(
                        xb.at[b], a_acc.at[dst_v.at[j]], txs[b])

                gather(0, 0).start()

                @pl.loop(0, nch, step=2)
                def _(j):
                    for b in (0, 1):
                        jj = j + b
                        gather(jj, b).wait()

                        @pl.when(jj > 0)
                        def _():
                            scat(jj - 1, 1 - b).wait()

                        scat(jj, b).start(add=True)

                        @pl.when(jj + 1 < nch)
                        def _():
                            gather(jj + 1, 1 - b).start()

                scat(nch - 1, 1).wait()
                plsc.subcore_barrier()
                pltpu.sync_copy(a_acc.at[pl.ds(row0, ROWS_PT)],
                                a_out.at[c, pl.ds(row0, ROWS_PT)])

            do_rel(xa, si1, di1, a1_out)
            do_rel(xa, si2, di2, a2_out)
            do_rel(xs, si3, di3, a3_out)

        bf16 = jnp.bfloat16
        return pl.kernel(
            body,
            out_type=[jax.ShapeDtypeStruct((NC, NPAD, D), bf16)] * 3,
            mesh=mesh,
            compiler_params=pltpu.CompilerParams(use_tc_tiling_on_sc=False),
            scratch_types=[
                pltpu.VMEM_SHARED((NPAD, D), bf16),
                pltpu.VMEM((NCHMAX, CHUNK), jnp.int32),
                pltpu.VMEM((NCHMAX, CHUNK), jnp.int32),
                pltpu.VMEM((2, CHUNK, D), bf16),
            ] + [pltpu.SemaphoreType.DMA] * 4,
        )

    def make_p2():
        def body(st1, dt1, ep1, st2, dt2, ep2, st3, dt3, ep3,
                 si1, di1, si2, di2, si3, di3, zb,
                 b1_out, b2_out, b3_out, sg1_o, dg1_o, sg2_o, dg2_o, sg3_o, dg3_o,
                 b_acc, src_v, dst_v, eb, sb, db, ge, gs, gd, te, ts, td):
            c = lax.axis_index("c")
            s = lax.axis_index("s")
            wid = c * NS + s
            row0 = s * ROWS_PT

            def do_rel(s_tab, d_tab, e_raw, src_i, dst_i, b_out, sg_out, dg_out):
                pltpu.sync_copy(src_i.at[wid], src_v)
                pltpu.sync_copy(dst_i.at[wid], dst_v)
                pltpu.sync_copy(zb, b_acc.at[pl.ds(row0, ROWS_PT)])
                plsc.subcore_barrier()

                def gathers(j, b):
                    erow = wid * EPTE + j * CHUNKE
                    return (
                        pltpu.make_async_copy(
                            e_raw.at[pl.ds(erow, CHUNKE)], eb.at[b], ge),
                        pltpu.make_async_copy(
                            s_tab.at[src_v.at[j]], sb.at[b], gs),
                        pltpu.make_async_copy(
                            d_tab.at[dst_v.at[j]], db.at[b], gd),
                    )

                def stores(j, b):
                    erow = wid * EPTE + j * CHUNKE
                    return (
                        (pltpu.make_async_copy(
                            eb.at[b], b_acc.at[dst_v.at[j]], te), True),
                        (pltpu.make_async_copy(
                            sb.at[b], sg_out.at[pl.ds(erow, CHUNKE)], ts), False),
                        (pltpu.make_async_copy(
                            db.at[b], dg_out.at[pl.ds(erow, CHUNKE)], td), False),
                    )

                for dsc in gathers(0, 0):
                    dsc.start()

                @pl.loop(0, NCHE, step=2)
                def _(j):
                    for b in (0, 1):
                        jj = j + b
                        for dsc in gathers(jj, b):
                            dsc.wait()

                        @pl.when(jj > 0)
                        def _():
                            for dsc, _add in stores(jj - 1, 1 - b):
                                dsc.wait()

                        for dsc, _add in stores(jj, b):
                            dsc.start(add=_add)

                        @pl.when(jj + 1 < NCHE)
                        def _():
                            for dsc in gathers(jj + 1, 1 - b):
                                dsc.start()

                for dsc, _add in stores(NCHE - 1, (NCHE - 1) & 1):
                    dsc.wait()
                plsc.subcore_barrier()
                pltpu.sync_copy(b_acc.at[pl.ds(row0, ROWS_PT)],
                                b_out.at[c, pl.ds(row0, ROWS_PT)])

            do_rel(st1, dt1, ep1, si1, di1, b1_out, sg1_o, dg1_o)
            do_rel(st2, dt2, ep2, si2, di2, b2_out, sg2_o, dg2_o)
            do_rel(st3, dt3, ep3, si3, di3, b3_out, sg3_o, dg3_o)

        return pl.kernel(
            body,
            out_type=[jax.ShapeDtypeStruct((NC, NPAD, DE), f32)] * 3
                     + [jax.ShapeDtypeStruct((E, DE), f32)] * 6,
            mesh=mesh,
            compiler_params=pltpu.CompilerParams(use_tc_tiling_on_sc=False),
            scratch_types=[
                pltpu.VMEM_SHARED((NPAD, DE), f32),
                pltpu.VMEM((NCHE, CHUNKE), jnp.int32),
                pltpu.VMEM((NCHE, CHUNKE), jnp.int32),
                pltpu.VMEM((2, CHUNKE, DE), f32),
                pltpu.VMEM((2, CHUNKE, DE), f32),
                pltpu.VMEM((2, CHUNKE, DE), f32),
            ] + [pltpu.SemaphoreType.DMA] * 6,
        )

    a1o, a2o, a3o = make_p1()(x_ap.astype(jnp.bfloat16),
                              x_sta.astype(jnp.bfloat16),
                              sx1, dx1, sx2, dx2, sx3, dx3,
                              jnp.zeros((ROWS_PT, D), jnp.bfloat16))
    (b1o, b2o, b3o, sg1, dg1, sg2, dg2, sg3, dg3) = make_p2()(
        s1, d1, e_apap, s2, d2, e_apsta, s3, d3, e_staap,
        se1, de1, se2, de2, se3, de3, zeros_b)

    a_spec = pl.BlockSpec((NC, blk, D), lambda i: (0, i, 0))
    b_spec = pl.BlockSpec((NC, blk, DE), lambda i: (0, i, 0))
    wdd = pl.BlockSpec((D, D), lambda i: (0, 0))
    wed = pl.BlockSpec((DE, D), lambda i: (0, 0))
    outD = pl.BlockSpec((blk, D), lambda i: (i, 0))
    nap, nsta = pl.pallas_call(
        _tc2_body,
        grid=(nb,),
        in_specs=[a_spec, b_spec, a_spec, b_spec, a_spec, b_spec,
                  row_spec, row_spec,
                  wdd, wed, wdd, wdd, wed, wdd, wdd, wed, wdd],
        out_specs=[outD, outD],
        out_shape=[jax.ShapeDtypeStruct((N, D), f32)] * 2,
    )(a1o, b1o, a2o, b2o, a3o, b3o, x_ap, x_sta,
      W_msg_apap, W_e_apap, W_self_apap,
      W_msg_apsta, W_e_apsta, W_self_apsta,
      W_msg_staap, W_e_staap, W_self_staap)

    E8 = E // 8
    eb = 2000
    neb = E8 // eb
    eg_spec = pl.BlockSpec((eb, D), lambda i: (i, 0))
    wee_spec = pl.BlockSpec((D, D), lambda i: (0, 0))
    eye8 = jnp.eye(8, dtype=f32)
    pk = lambda a: a.reshape(E8, D)
    eo1, eo2, eo3 = pl.pallas_call(
        _tc3_body,
        grid=(neb,),
        in_specs=[eg_spec] * 9 + [wee_spec] * 3,
        out_specs=[eg_spec] * 3,
        out_shape=[jax.ShapeDtypeStruct((E8, D), f32)] * 3,
    )(pk(sg1), pk(dg1), pk(e_apap), pk(sg2), pk(dg2), pk(e_apsta),
      pk(sg3), pk(dg3), pk(e_staap),
      jnp.kron(eye8, W_ee_apap), jnp.kron(eye8, W_ee_apsta),
      jnp.kron(eye8, W_ee_staap))

    return (nap, nsta, eo1.reshape(E, DE), eo2.reshape(E, DE),
            eo3.reshape(E, DE))
